# baseline (speedup 1.0000x reference)
import functools

import jax
import jax.numpy as jnp
import numpy as np
from jax import lax
from jax.experimental import pallas as pl
from jax.experimental.pallas import tpu as pltpu
from jax.experimental.pallas import tpu_sc as plsc

IN_CH = 128
OUT_CH = 128
SC_CH = 32
N = 10000
KP = 15
KNN = 16
RADIUS = 0.1
KP_EXTENT = 2.0
CONV_RADIUS = 2.5
EXTENT = RADIUS * KP_EXTENT / CONV_RADIUS
R2 = RADIUS * RADIUS

NPAD = 10240
RB = 64
GB = 256
TBL_W = 128
SC_CHUNK = 128


def _kp_np():
    rng = np.random.RandomState(42)
    kp = rng.randn(KP, 3).astype(np.float32)
    kp /= (np.linalg.norm(kp, axis=1, keepdims=True) + 1e-9)
    kp *= rng.rand(KP, 1).astype(np.float32)
    kp[0] = 0.0
    return (kp * EXTENT).astype(np.float32)


_KPTS = _kp_np()


def _convin_body(x_ref, w_ref, b_ref, o_ref):
    o_ref[...] = (
        jnp.dot(x_ref[...], w_ref[...], preferred_element_type=jnp.float32)
        + b_ref[...]
    )


def _conv_in(x, wT, b2):
    return pl.pallas_call(
        _convin_body,
        grid=(NPAD // 1024,),
        in_specs=[
            pl.BlockSpec((1024, IN_CH), lambda i: (i, 0)),
            pl.BlockSpec((IN_CH, SC_CH), lambda i: (0, 0)),
            pl.BlockSpec((1, SC_CH), lambda i: (0, 0)),
        ],
        out_specs=pl.BlockSpec((1024, SC_CH), lambda i: (i, 0)),
        out_shape=jax.ShapeDtypeStruct((NPAD, SC_CH), jnp.float32),
    )(x, wT, b2)


NG = 128
GS = NPAD // NG


def _knn_body(cr_ref, ct_ref, o_ref):
    cr = cr_ref[...]
    ct = ct_ref[...]
    sqr = jnp.sum(cr * cr, axis=1, keepdims=True)
    sqc = jnp.sum(ct * ct, axis=0, keepdims=True)
    d2 = sqr + sqc - 2.0 * jnp.dot(cr, ct, preferred_element_type=jnp.float32)

    g = d2[:, :NG]
    for k in range(1, GS):
        g = jnp.minimum(g, d2[:, k * NG:(k + 1) * NG])
    lane = lax.broadcasted_iota(jnp.int32, (RB, NG), 1)
    gsel = []
    for _ in range(KNN):
        m = jnp.min(g, axis=1, keepdims=True)
        cand = jnp.where(g <= m, lane, jnp.int32(2**30))
        am = jnp.min(cand, axis=1, keepdims=True)
        gsel.append(am)
        g = jnp.where(cand == am, jnp.float32(1e30), g)
    g16 = jnp.concatenate(gsel, axis=1)

    idx128 = jnp.tile(g16, (1, NG // KNN))
    sec = lane // KNN
    cs, js = [], []
    for t in range(GS // 8):
        acc = None
        for s in range(8):
            k = 8 * t + s
            gth = jnp.take_along_axis(d2[:, k * NG:(k + 1) * NG], idx128, axis=1)
            acc = gth if acc is None else jnp.where(sec == s, gth, acc)
        cs.append(acc)
        js.append(NG * (8 * t + sec) + idx128)
    c = jnp.concatenate(cs, axis=1)
    ji = jnp.concatenate(js, axis=1)

    g2 = c[:, :NG]
    for k in range(1, GS // 8):
        g2 = jnp.minimum(g2, c[:, k * NG:(k + 1) * NG])
    hsel = []
    for _ in range(KNN):
        m = jnp.min(g2, axis=1, keepdims=True)
        cand = jnp.where(g2 <= m, lane, jnp.int32(2**30))
        am = jnp.min(cand, axis=1, keepdims=True)
        hsel.append(am)
        g2 = jnp.where(cand == am, jnp.float32(1e30), g2)
    h16 = jnp.concatenate(hsel, axis=1)
    hdx128 = jnp.tile(h16, (1, NG // KNN))
    cs2, js2 = [], []
    for t in range(2):
        accv = acci = None
        for s in range(8):
            k = 8 * t + s
            if k < GS // 8:
                gv = jnp.take_along_axis(c[:, k * NG:(k + 1) * NG], hdx128, axis=1)
                gi = jnp.take_along_axis(ji[:, k * NG:(k + 1) * NG], hdx128, axis=1)
            else:
                gv = jnp.full((RB, NG), 1e30, jnp.float32)
                gi = jnp.full((RB, NG), 2**30, jnp.int32)
            accv = gv if accv is None else jnp.where(sec == s, gv, accv)
            acci = gi if acci is None else jnp.where(sec == s, gi, acci)
        cs2.append(accv)
        js2.append(acci)
    c2 = jnp.concatenate(cs2, axis=1)
    j2 = jnp.concatenate(js2, axis=1)

    idxs = []
    for _ in range(KNN):
        m = jnp.min(c2, axis=1, keepdims=True)
        cand = jnp.where(c2 <= m, j2, jnp.int32(2**30))
        am = jnp.min(cand, axis=1, keepdims=True)
        idxs.append(am)
        c2 = jnp.where(cand == am, jnp.float32(1e30), c2)
    o_ref[...] = jnp.concatenate(idxs, axis=1)


def _knn(cp, ct):
    return pl.pallas_call(
        _knn_body,
        grid=(NPAD // RB,),
        in_specs=[
            pl.BlockSpec((RB, 8), lambda i: (i, 0)),
            pl.BlockSpec((8, NPAD), lambda i: (0, 0)),
        ],
        out_specs=pl.BlockSpec((RB, KNN), lambda i: (i, 0)),
        out_shape=jax.ShapeDtypeStruct((NPAD, KNN), jnp.int32),
    )(cp, ct)


SC_B = NPAD * KNN
SC_NW = 32


def _gather_sc(idx_flat, table):
    mesh = plsc.VectorSubcoreMesh(core_axis_name="c", subcore_axis_name="s")
    bpw = SC_B // SC_NW
    nch = bpw // SC_CHUNK

    @functools.partial(
        pl.kernel,
        out_type=jax.ShapeDtypeStruct((SC_B, TBL_W), jnp.float32),
        mesh=mesh,
        scratch_types=[
            pltpu.VMEM((SC_CHUNK,), jnp.int32),
            pltpu.VMEM((SC_CHUNK, TBL_W), jnp.float32),
            pltpu.SemaphoreType.DMA,
        ],
    )
    def gk(idx_hbm, tbl_hbm, out_hbm, idx_v, rows_v, sem):
        wid = lax.axis_index("s") * 2 + lax.axis_index("c")
        base = wid * bpw

        def body(j, carry):
            off = pl.multiple_of(base + j * SC_CHUNK, 8)
            pltpu.sync_copy(idx_hbm.at[pl.ds(off, SC_CHUNK)], idx_v)
            pltpu.async_copy(tbl_hbm.at[idx_v], rows_v, sem).wait()
            pltpu.sync_copy(rows_v, out_hbm.at[pl.ds(off, SC_CHUNK)])
            return carry

        lax.fori_loop(0, nch, body, 0)

    return gk(idx_flat, table)


GB2 = 2048
PW = 512

_KP16 = np.pad(_KPTS, ((0, 1), (0, 0)))
_SEL = (np.arange(PW)[None, :] // SC_CH == np.arange(KNN)[:, None]).astype(np.float32)
_FOLD = (np.arange(PW)[:, None] % SC_CH == np.arange(SC_CH)[None, :]).astype(np.float32)
_KC = np.zeros((8, PW), np.float32)
_KC[0, :KNN] = _KP16[:, 0]
_KC[1, :KNN] = _KP16[:, 1]
_KC[2, :KNN] = _KP16[:, 2]
_KC[3, :KP] = 1.0


def _agg_body(g_ref, cq_ref, w2_ref, fw_ref, bo_ref, kc_ref, sel_ref, y_ref):
    g = g_ref[...]
    nf = g[:, :SC_CH]
    f2 = jnp.dot(nf, w2_ref[...], preferred_element_type=jnp.float32)
    rx = g[:, SC_CH + 0:SC_CH + 1] - cq_ref[:, 0:1]
    ry = g[:, SC_CH + 1:SC_CH + 2] - cq_ref[:, 1:2]
    rz = g[:, SC_CH + 2:SC_CH + 3] - cq_ref[:, 2:3]
    nd2 = rx * rx + ry * ry + rz * rz
    maskf = (nd2 <= R2).astype(jnp.float32)
    dx = rx - kc_ref[0:1, :KNN]
    dy = ry - kc_ref[1:2, :KNN]
    dz = rz - kc_ref[2:3, :KNN]
    dist = jnp.sqrt(dx * dx + dy * dy + dz * dz + 1e-12)
    infl = jnp.maximum(0.0, 1.0 - dist / EXTENT) * maskf * kc_ref[3:4, :KNN]
    inflb = jnp.dot(infl, sel_ref[...], preferred_element_type=jnp.float32)
    z = inflb * f2
    zq = jnp.sum(z.reshape(GB2 // KNN, KNN, PW), axis=1)
    y_ref[...] = (
        jnp.dot(zq, fw_ref[...], preferred_element_type=jnp.float32)
        + bo_ref[...]
    )


def _agg(g2d, crep, w2all, fw, bo2):
    return pl.pallas_call(
        _agg_body,
        grid=(SC_B // GB2,),
        in_specs=[
            pl.BlockSpec((GB2, TBL_W), lambda i: (i, 0)),
            pl.BlockSpec((GB2, 8), lambda i: (i, 0)),
            pl.BlockSpec((SC_CH, PW), lambda i: (0, 0)),
            pl.BlockSpec((PW, OUT_CH), lambda i: (0, 0)),
            pl.BlockSpec((1, OUT_CH), lambda i: (0, 0)),
            pl.BlockSpec((8, PW), lambda i: (0, 0)),
            pl.BlockSpec((KNN, PW), lambda i: (0, 0)),
        ],
        out_specs=pl.BlockSpec((GB2 // KNN, OUT_CH), lambda i: (i, 0)),
        out_shape=jax.ShapeDtypeStruct((NPAD, OUT_CH), jnp.float32),
    )(g2d, crep, w2all, fw, bo2, jnp.asarray(_KC), jnp.asarray(_SEL))


def _bn_body(y_ref, g_ref, b_ref, o_ref):
    y = y_ref[...]
    rows = lax.broadcasted_iota(jnp.int32, (NPAD, 1), 0)
    validf = (rows < N).astype(jnp.float32)
    yv = y * validf
    mean = jnp.sum(yv, axis=0, keepdims=True) / N
    msq = jnp.sum(yv * yv, axis=0, keepdims=True) / N
    var = msq - mean * mean
    xh = (y - mean) / jnp.sqrt(var + 1e-5)
    out = xh * g_ref[...] + b_ref[...]
    o_ref[...] = jnp.where(out >= 0, out, 0.1 * out)


def _bn_lrelu(y, g2, b2):
    return pl.pallas_call(
        _bn_body,
        out_shape=jax.ShapeDtypeStruct((NPAD, OUT_CH), jnp.float32),
    )(y, g2, b2)


def _cloud(x, coords, w_inT, b_in2, w2all, fw, bo2, g2, be2):
    xt = jnp.pad(jnp.transpose(x[0]), ((0, NPAD - N), (0, 0)))
    cpad = jnp.concatenate(
        [coords[0], jnp.full((NPAD - N, 3), 1e3, jnp.float32)], axis=0
    )
    cp = jnp.pad(cpad, ((0, 0), (0, 5)))
    ct = jnp.transpose(cp)

    feats = _conv_in(xt, w_inT, b_in2)
    idx = _knn(cp, ct)
    table = jnp.concatenate(
        [feats, cp[:, :3], jnp.zeros((NPAD, TBL_W - SC_CH - 3), jnp.float32)],
        axis=1,
    )
    g = _gather_sc(idx.reshape(-1), table)
    crep = jnp.repeat(cp, KNN, axis=0)
    y = _agg(g, crep, w2all, fw, bo2)
    out = _bn_lrelu(y, g2, be2)
    return jnp.transpose(out[:N])[None]


def kernel(src, tgt, src_coords, tgt_coords, W_in, b_in, W_kp, W_out, b_out, gamma, beta):
    w_inT = jnp.transpose(W_in)
    b_in2 = b_in[None, :]
    w2all = jnp.pad(
        jnp.transpose(W_kp, (1, 0, 2)).reshape(SC_CH, KP * SC_CH),
        ((0, 0), (0, PW - KP * SC_CH)),
    )
    fw = jnp.dot(jnp.asarray(_FOLD), jnp.transpose(W_out))
    bo2 = b_out[None, :]
    g2 = gamma[None, :]
    be2 = beta[None, :]
    s3 = _cloud(src, src_coords, w_inT, b_in2, w2all, fw, bo2, g2, be2)
    t3 = _cloud(tgt, tgt_coords, w_inT, b_in2, w2all, fw, bo2, g2, be2)
    return (s3, t3, src_coords, tgt_coords)


# speedup vs baseline: 2.4208x; 2.4208x over previous
import functools

import jax
import jax.numpy as jnp
import numpy as np
from jax import lax
from jax.experimental import pallas as pl
from jax.experimental.pallas import tpu as pltpu
from jax.experimental.pallas import tpu_sc as plsc

IN_CH = 128
OUT_CH = 128
SC_CH = 32
N = 10000
KP = 15
KNN = 16
RADIUS = 0.1
KP_EXTENT = 2.0
CONV_RADIUS = 2.5
EXTENT = RADIUS * KP_EXTENT / CONV_RADIUS
R2 = RADIUS * RADIUS

NPAD = 10240
RB = 256
GB = 256
TBL_W = 128
SC_CHUNK = 128


def _kp_np():
    rng = np.random.RandomState(42)
    kp = rng.randn(KP, 3).astype(np.float32)
    kp /= (np.linalg.norm(kp, axis=1, keepdims=True) + 1e-9)
    kp *= rng.rand(KP, 1).astype(np.float32)
    kp[0] = 0.0
    return (kp * EXTENT).astype(np.float32)


_KPTS = _kp_np()


def _convin_body(x_ref, w_ref, b_ref, o_ref):
    o_ref[...] = (
        jnp.dot(x_ref[...], w_ref[...], preferred_element_type=jnp.float32)
        + b_ref[...]
    )


def _conv_in(x, wT, b2):
    return pl.pallas_call(
        _convin_body,
        grid=(NPAD // 1024,),
        in_specs=[
            pl.BlockSpec((1024, IN_CH), lambda i: (i, 0)),
            pl.BlockSpec((IN_CH, SC_CH), lambda i: (0, 0)),
            pl.BlockSpec((1, SC_CH), lambda i: (0, 0)),
        ],
        out_specs=pl.BlockSpec((1024, SC_CH), lambda i: (i, 0)),
        out_shape=jax.ShapeDtypeStruct((NPAD, SC_CH), jnp.float32),
    )(x, wT, b2)


NG = 128
GS = NPAD // NG


def _knn_body(cr_ref, ct_ref, o_ref):
    cr = cr_ref[...]
    ct = ct_ref[...]
    sqr = jnp.sum(cr * cr, axis=1, keepdims=True)
    sqc = jnp.sum(ct * ct, axis=0, keepdims=True)
    d2 = sqr + sqc - 2.0 * jnp.dot(cr, ct, preferred_element_type=jnp.float32)

    g = d2[:, :NG]
    for k in range(1, GS):
        g = jnp.minimum(g, d2[:, k * NG:(k + 1) * NG])
    lane = lax.broadcasted_iota(jnp.int32, (RB, NG), 1)
    gsel = []
    for _ in range(KNN):
        m = jnp.min(g, axis=1, keepdims=True)
        cand = jnp.where(g <= m, lane, jnp.int32(2**30))
        am = jnp.min(cand, axis=1, keepdims=True)
        gsel.append(am)
        g = jnp.where(cand == am, jnp.float32(1e30), g)
    g16 = jnp.concatenate(gsel, axis=1)

    idx128 = jnp.tile(g16, (1, NG // KNN))
    sec = lane // KNN
    cs, js = [], []
    for t in range(GS // 8):
        acc = None
        for s in range(8):
            k = 8 * t + s
            gth = jnp.take_along_axis(d2[:, k * NG:(k + 1) * NG], idx128, axis=1)
            acc = gth if acc is None else jnp.where(sec == s, gth, acc)
        cs.append(acc)
        js.append(NG * (8 * t + sec) + idx128)
    c = jnp.concatenate(cs, axis=1)
    ji = jnp.concatenate(js, axis=1)
    idxs = []
    for _ in range(KNN):
        m = jnp.min(c, axis=1, keepdims=True)
        cand = jnp.where(c <= m, ji, jnp.int32(2**30))
        am = jnp.min(cand, axis=1, keepdims=True)
        idxs.append(am)
        c = jnp.where(cand == am, jnp.float32(1e30), c)
    o_ref[...] = jnp.concatenate(idxs, axis=1)


def _knn(cp, ct):
    return pl.pallas_call(
        _knn_body,
        grid=(NPAD // RB,),
        in_specs=[
            pl.BlockSpec((RB, 8), lambda i: (i, 0)),
            pl.BlockSpec((8, NPAD), lambda i: (0, 0)),
        ],
        out_specs=pl.BlockSpec((RB, KNN), lambda i: (i, 0)),
        out_shape=jax.ShapeDtypeStruct((NPAD, KNN), jnp.int32),
    )(cp, ct)


SC_B = NPAD * KNN
SC_NW = 32


def _gather_sc(idx_flat, table):
    mesh = plsc.VectorSubcoreMesh(core_axis_name="c", subcore_axis_name="s")
    bpw = SC_B // SC_NW
    nch = bpw // SC_CHUNK

    @functools.partial(
        pl.kernel,
        out_type=jax.ShapeDtypeStruct((SC_B, TBL_W), jnp.float32),
        mesh=mesh,
        scratch_types=[
            pltpu.VMEM((SC_CHUNK,), jnp.int32),
            pltpu.VMEM((SC_CHUNK, TBL_W), jnp.float32),
            pltpu.SemaphoreType.DMA,
        ],
    )
    def gk(idx_hbm, tbl_hbm, out_hbm, idx_v, rows_v, sem):
        wid = lax.axis_index("s") * 2 + lax.axis_index("c")
        base = wid * bpw

        def body(j, carry):
            off = pl.multiple_of(base + j * SC_CHUNK, 8)
            pltpu.sync_copy(idx_hbm.at[pl.ds(off, SC_CHUNK)], idx_v)
            pltpu.async_copy(tbl_hbm.at[idx_v], rows_v, sem).wait()
            pltpu.sync_copy(rows_v, out_hbm.at[pl.ds(off, SC_CHUNK)])
            return carry

        lax.fori_loop(0, nch, body, 0)

    return gk(idx_flat, table)


GB2 = 2048
PW = 512

_KP16 = np.pad(_KPTS, ((0, 1), (0, 0)))
_SEL = (np.arange(PW)[None, :] // SC_CH == np.arange(KNN)[:, None]).astype(np.float32)
_FOLD = (np.arange(PW)[:, None] % SC_CH == np.arange(SC_CH)[None, :]).astype(np.float32)
_KC = np.zeros((8, PW), np.float32)
_KC[0, :KNN] = _KP16[:, 0]
_KC[1, :KNN] = _KP16[:, 1]
_KC[2, :KNN] = _KP16[:, 2]
_KC[3, :KP] = 1.0


def _agg_body(g_ref, cq_ref, w2_ref, fw_ref, bo_ref, kc_ref, sel_ref, y_ref):
    g = g_ref[...]
    nf = g[:, :SC_CH]
    f2 = jnp.dot(nf, w2_ref[...], preferred_element_type=jnp.float32)
    rx = g[:, SC_CH + 0:SC_CH + 1] - cq_ref[:, 0:1]
    ry = g[:, SC_CH + 1:SC_CH + 2] - cq_ref[:, 1:2]
    rz = g[:, SC_CH + 2:SC_CH + 3] - cq_ref[:, 2:3]
    nd2 = rx * rx + ry * ry + rz * rz
    maskf = (nd2 <= R2).astype(jnp.float32)
    dx = rx - kc_ref[0:1, :KNN]
    dy = ry - kc_ref[1:2, :KNN]
    dz = rz - kc_ref[2:3, :KNN]
    dist = jnp.sqrt(dx * dx + dy * dy + dz * dz + 1e-12)
    infl = jnp.maximum(0.0, 1.0 - dist / EXTENT) * maskf * kc_ref[3:4, :KNN]
    inflb = jnp.dot(infl, sel_ref[...], preferred_element_type=jnp.float32)
    z = inflb * f2
    zq = jnp.sum(z.reshape(GB2 // KNN, KNN, PW), axis=1)
    y_ref[...] = (
        jnp.dot(zq, fw_ref[...], preferred_element_type=jnp.float32)
        + bo_ref[...]
    )


def _agg(g2d, crep, w2all, fw, bo2):
    return pl.pallas_call(
        _agg_body,
        grid=(SC_B // GB2,),
        in_specs=[
            pl.BlockSpec((GB2, TBL_W), lambda i: (i, 0)),
            pl.BlockSpec((GB2, 8), lambda i: (i, 0)),
            pl.BlockSpec((SC_CH, PW), lambda i: (0, 0)),
            pl.BlockSpec((PW, OUT_CH), lambda i: (0, 0)),
            pl.BlockSpec((1, OUT_CH), lambda i: (0, 0)),
            pl.BlockSpec((8, PW), lambda i: (0, 0)),
            pl.BlockSpec((KNN, PW), lambda i: (0, 0)),
        ],
        out_specs=pl.BlockSpec((GB2 // KNN, OUT_CH), lambda i: (i, 0)),
        out_shape=jax.ShapeDtypeStruct((NPAD, OUT_CH), jnp.float32),
    )(g2d, crep, w2all, fw, bo2, jnp.asarray(_KC), jnp.asarray(_SEL))


def _bn_body(y_ref, g_ref, b_ref, o_ref):
    y = y_ref[...]
    rows = lax.broadcasted_iota(jnp.int32, (NPAD, 1), 0)
    validf = (rows < N).astype(jnp.float32)
    yv = y * validf
    mean = jnp.sum(yv, axis=0, keepdims=True) / N
    msq = jnp.sum(yv * yv, axis=0, keepdims=True) / N
    var = msq - mean * mean
    xh = (y - mean) / jnp.sqrt(var + 1e-5)
    out = xh * g_ref[...] + b_ref[...]
    o_ref[...] = jnp.where(out >= 0, out, 0.1 * out)


def _bn_lrelu(y, g2, b2):
    return pl.pallas_call(
        _bn_body,
        out_shape=jax.ShapeDtypeStruct((NPAD, OUT_CH), jnp.float32),
    )(y, g2, b2)


def _cloud(x, coords, w_inT, b_in2, w2all, fw, bo2, g2, be2):
    xt = jnp.pad(jnp.transpose(x[0]), ((0, NPAD - N), (0, 0)))
    cpad = jnp.concatenate(
        [coords[0], jnp.full((NPAD - N, 3), 1e3, jnp.float32)], axis=0
    )
    cp = jnp.pad(cpad, ((0, 0), (0, 5)))
    ct = jnp.transpose(cp)

    feats = _conv_in(xt, w_inT, b_in2)
    idx = _knn(cp, ct)
    table = jnp.concatenate(
        [feats, cp[:, :3], jnp.zeros((NPAD, TBL_W - SC_CH - 3), jnp.float32)],
        axis=1,
    )
    g = _gather_sc(idx.reshape(-1), table)
    crep = jnp.repeat(cp, KNN, axis=0)
    y = _agg(g, crep, w2all, fw, bo2)
    out = _bn_lrelu(y, g2, be2)
    return jnp.transpose(out[:N])[None]


def kernel(src, tgt, src_coords, tgt_coords, W_in, b_in, W_kp, W_out, b_out, gamma, beta):
    w_inT = jnp.transpose(W_in)
    b_in2 = b_in[None, :]
    w2all = jnp.pad(
        jnp.transpose(W_kp, (1, 0, 2)).reshape(SC_CH, KP * SC_CH),
        ((0, 0), (0, PW - KP * SC_CH)),
    )
    fw = jnp.dot(jnp.asarray(_FOLD), jnp.transpose(W_out))
    bo2 = b_out[None, :]
    g2 = gamma[None, :]
    be2 = beta[None, :]
    s3 = _cloud(src, src_coords, w_inT, b_in2, w2all, fw, bo2, g2, be2)
    t3 = _cloud(tgt, tgt_coords, w_inT, b_in2, w2all, fw, bo2, g2, be2)
    return (s3, t3, src_coords, tgt_coords)


# speedup vs baseline: 2.4398x; 1.0078x over previous
import functools

import jax
import jax.numpy as jnp
import numpy as np
from jax import lax
from jax.experimental import pallas as pl
from jax.experimental.pallas import tpu as pltpu
from jax.experimental.pallas import tpu_sc as plsc

IN_CH = 128
OUT_CH = 128
SC_CH = 32
N = 10000
KP = 15
KNN = 16
RADIUS = 0.1
KP_EXTENT = 2.0
CONV_RADIUS = 2.5
EXTENT = RADIUS * KP_EXTENT / CONV_RADIUS
R2 = RADIUS * RADIUS

NPAD = 10240
RB = 512
GB = 256
TBL_W = 128
SC_CHUNK = 128


def _kp_np():
    rng = np.random.RandomState(42)
    kp = rng.randn(KP, 3).astype(np.float32)
    kp /= (np.linalg.norm(kp, axis=1, keepdims=True) + 1e-9)
    kp *= rng.rand(KP, 1).astype(np.float32)
    kp[0] = 0.0
    return (kp * EXTENT).astype(np.float32)


_KPTS = _kp_np()


def _convin_body(x_ref, w_ref, b_ref, o_ref):
    o_ref[...] = (
        jnp.dot(x_ref[...], w_ref[...], preferred_element_type=jnp.float32)
        + b_ref[...]
    )


def _conv_in(x, wT, b2):
    return pl.pallas_call(
        _convin_body,
        grid=(NPAD // 1024,),
        in_specs=[
            pl.BlockSpec((1024, IN_CH), lambda i: (i, 0)),
            pl.BlockSpec((IN_CH, SC_CH), lambda i: (0, 0)),
            pl.BlockSpec((1, SC_CH), lambda i: (0, 0)),
        ],
        out_specs=pl.BlockSpec((1024, SC_CH), lambda i: (i, 0)),
        out_shape=jax.ShapeDtypeStruct((NPAD, SC_CH), jnp.float32),
    )(x, wT, b2)


NG = 128
GS = NPAD // NG


def _knn_body(cr_ref, ct_ref, o_ref):
    cr = cr_ref[...]
    ct = ct_ref[...]
    sqr = jnp.sum(cr * cr, axis=1, keepdims=True)
    sqc = jnp.sum(ct * ct, axis=0, keepdims=True)
    d2 = sqr + sqc - 2.0 * jnp.dot(cr, ct, preferred_element_type=jnp.float32)

    g = d2[:, :NG]
    for k in range(1, GS):
        g = jnp.minimum(g, d2[:, k * NG:(k + 1) * NG])
    lane = lax.broadcasted_iota(jnp.int32, (RB, NG), 1)
    gsel = []
    for _ in range(KNN):
        m = jnp.min(g, axis=1, keepdims=True)
        cand = jnp.where(g <= m, lane, jnp.int32(2**30))
        am = jnp.min(cand, axis=1, keepdims=True)
        gsel.append(am)
        g = jnp.where(cand == am, jnp.float32(1e30), g)
    g16 = jnp.concatenate(gsel, axis=1)

    idx128 = jnp.tile(g16, (1, NG // KNN))
    sec = lane // KNN
    cs, js = [], []
    for t in range(GS // 8):
        acc = None
        for s in range(8):
            k = 8 * t + s
            gth = jnp.take_along_axis(d2[:, k * NG:(k + 1) * NG], idx128, axis=1)
            acc = gth if acc is None else jnp.where(sec == s, gth, acc)
        cs.append(acc)
        js.append(NG * (8 * t + sec) + idx128)
    c = jnp.concatenate(cs, axis=1)
    ji = jnp.concatenate(js, axis=1)
    idxs = []
    for _ in range(KNN):
        m = jnp.min(c, axis=1, keepdims=True)
        cand = jnp.where(c <= m, ji, jnp.int32(2**30))
        am = jnp.min(cand, axis=1, keepdims=True)
        idxs.append(am)
        c = jnp.where(cand == am, jnp.float32(1e30), c)
    o_ref[...] = jnp.concatenate(idxs, axis=1)


def _knn(cp, ct):
    return pl.pallas_call(
        _knn_body,
        grid=(NPAD // RB,),
        in_specs=[
            pl.BlockSpec((RB, 8), lambda i: (i, 0)),
            pl.BlockSpec((8, NPAD), lambda i: (0, 0)),
        ],
        out_specs=pl.BlockSpec((RB, KNN), lambda i: (i, 0)),
        out_shape=jax.ShapeDtypeStruct((NPAD, KNN), jnp.int32),
    )(cp, ct)


SC_B = NPAD * KNN
SC_NW = 32


def _gather_sc(idx_flat, table):
    mesh = plsc.VectorSubcoreMesh(core_axis_name="c", subcore_axis_name="s")
    bpw = SC_B // SC_NW
    nch = bpw // SC_CHUNK

    @functools.partial(
        pl.kernel,
        out_type=jax.ShapeDtypeStruct((SC_B, TBL_W), jnp.float32),
        mesh=mesh,
        scratch_types=[
            pltpu.VMEM((SC_CHUNK,), jnp.int32),
            pltpu.VMEM((SC_CHUNK, TBL_W), jnp.float32),
            pltpu.SemaphoreType.DMA,
        ],
    )
    def gk(idx_hbm, tbl_hbm, out_hbm, idx_v, rows_v, sem):
        wid = lax.axis_index("s") * 2 + lax.axis_index("c")
        base = wid * bpw

        def body(j, carry):
            off = pl.multiple_of(base + j * SC_CHUNK, 8)
            pltpu.sync_copy(idx_hbm.at[pl.ds(off, SC_CHUNK)], idx_v)
            pltpu.async_copy(tbl_hbm.at[idx_v], rows_v, sem).wait()
            pltpu.sync_copy(rows_v, out_hbm.at[pl.ds(off, SC_CHUNK)])
            return carry

        lax.fori_loop(0, nch, body, 0)

    return gk(idx_flat, table)


GB2 = 2048
PW = 512

_KP16 = np.pad(_KPTS, ((0, 1), (0, 0)))
_SEL = (np.arange(PW)[None, :] // SC_CH == np.arange(KNN)[:, None]).astype(np.float32)
_FOLD = (np.arange(PW)[:, None] % SC_CH == np.arange(SC_CH)[None, :]).astype(np.float32)
_KC = np.zeros((8, PW), np.float32)
_KC[0, :KNN] = _KP16[:, 0]
_KC[1, :KNN] = _KP16[:, 1]
_KC[2, :KNN] = _KP16[:, 2]
_KC[3, :KP] = 1.0


def _agg_body(g_ref, cq_ref, w2_ref, fw_ref, bo_ref, kc_ref, sel_ref, y_ref):
    g = g_ref[...]
    nf = g[:, :SC_CH]
    f2 = jnp.dot(nf, w2_ref[...], preferred_element_type=jnp.float32)
    rx = g[:, SC_CH + 0:SC_CH + 1] - cq_ref[:, 0:1]
    ry = g[:, SC_CH + 1:SC_CH + 2] - cq_ref[:, 1:2]
    rz = g[:, SC_CH + 2:SC_CH + 3] - cq_ref[:, 2:3]
    nd2 = rx * rx + ry * ry + rz * rz
    maskf = (nd2 <= R2).astype(jnp.float32)
    dx = rx - kc_ref[0:1, :KNN]
    dy = ry - kc_ref[1:2, :KNN]
    dz = rz - kc_ref[2:3, :KNN]
    dist = jnp.sqrt(dx * dx + dy * dy + dz * dz + 1e-12)
    infl = jnp.maximum(0.0, 1.0 - dist / EXTENT) * maskf * kc_ref[3:4, :KNN]
    inflb = jnp.dot(infl, sel_ref[...], preferred_element_type=jnp.float32)
    z = inflb * f2
    zq = jnp.sum(z.reshape(GB2 // KNN, KNN, PW), axis=1)
    y_ref[...] = (
        jnp.dot(zq, fw_ref[...], preferred_element_type=jnp.float32)
        + bo_ref[...]
    )


def _agg(g2d, crep, w2all, fw, bo2):
    return pl.pallas_call(
        _agg_body,
        grid=(SC_B // GB2,),
        in_specs=[
            pl.BlockSpec((GB2, TBL_W), lambda i: (i, 0)),
            pl.BlockSpec((GB2, 8), lambda i: (i, 0)),
            pl.BlockSpec((SC_CH, PW), lambda i: (0, 0)),
            pl.BlockSpec((PW, OUT_CH), lambda i: (0, 0)),
            pl.BlockSpec((1, OUT_CH), lambda i: (0, 0)),
            pl.BlockSpec((8, PW), lambda i: (0, 0)),
            pl.BlockSpec((KNN, PW), lambda i: (0, 0)),
        ],
        out_specs=pl.BlockSpec((GB2 // KNN, OUT_CH), lambda i: (i, 0)),
        out_shape=jax.ShapeDtypeStruct((NPAD, OUT_CH), jnp.float32),
    )(g2d, crep, w2all, fw, bo2, jnp.asarray(_KC), jnp.asarray(_SEL))


def _bn_body(y_ref, g_ref, b_ref, o_ref):
    y = y_ref[...]
    rows = lax.broadcasted_iota(jnp.int32, (NPAD, 1), 0)
    validf = (rows < N).astype(jnp.float32)
    yv = y * validf
    mean = jnp.sum(yv, axis=0, keepdims=True) / N
    msq = jnp.sum(yv * yv, axis=0, keepdims=True) / N
    var = msq - mean * mean
    xh = (y - mean) / jnp.sqrt(var + 1e-5)
    out = xh * g_ref[...] + b_ref[...]
    o_ref[...] = jnp.where(out >= 0, out, 0.1 * out)


def _bn_lrelu(y, g2, b2):
    return pl.pallas_call(
        _bn_body,
        out_shape=jax.ShapeDtypeStruct((NPAD, OUT_CH), jnp.float32),
    )(y, g2, b2)


def _cloud(x, coords, w_inT, b_in2, w2all, fw, bo2, g2, be2):
    xt = jnp.pad(jnp.transpose(x[0]), ((0, NPAD - N), (0, 0)))
    cpad = jnp.concatenate(
        [coords[0], jnp.full((NPAD - N, 3), 1e3, jnp.float32)], axis=0
    )
    cp = jnp.pad(cpad, ((0, 0), (0, 5)))
    ct = jnp.transpose(cp)

    feats = _conv_in(xt, w_inT, b_in2)
    idx = _knn(cp, ct)
    table = jnp.concatenate(
        [feats, cp[:, :3], jnp.zeros((NPAD, TBL_W - SC_CH - 3), jnp.float32)],
        axis=1,
    )
    g = _gather_sc(idx.reshape(-1), table)
    crep = jnp.repeat(cp, KNN, axis=0)
    y = _agg(g, crep, w2all, fw, bo2)
    out = _bn_lrelu(y, g2, be2)
    return jnp.transpose(out[:N])[None]


def kernel(src, tgt, src_coords, tgt_coords, W_in, b_in, W_kp, W_out, b_out, gamma, beta):
    w_inT = jnp.transpose(W_in)
    b_in2 = b_in[None, :]
    w2all = jnp.pad(
        jnp.transpose(W_kp, (1, 0, 2)).reshape(SC_CH, KP * SC_CH),
        ((0, 0), (0, PW - KP * SC_CH)),
    )
    fw = jnp.dot(jnp.asarray(_FOLD), jnp.transpose(W_out))
    bo2 = b_out[None, :]
    g2 = gamma[None, :]
    be2 = beta[None, :]
    s3 = _cloud(src, src_coords, w_inT, b_in2, w2all, fw, bo2, g2, be2)
    t3 = _cloud(tgt, tgt_coords, w_inT, b_in2, w2all, fw, bo2, g2, be2)
    return (s3, t3, src_coords, tgt_coords)


# speedup vs baseline: 2.4554x; 1.0064x over previous
import functools

import jax
import jax.numpy as jnp
import numpy as np
from jax import lax
from jax.experimental import pallas as pl
from jax.experimental.pallas import tpu as pltpu
from jax.experimental.pallas import tpu_sc as plsc

IN_CH = 128
OUT_CH = 128
SC_CH = 32
N = 10000
KP = 15
KNN = 16
RADIUS = 0.1
KP_EXTENT = 2.0
CONV_RADIUS = 2.5
EXTENT = RADIUS * KP_EXTENT / CONV_RADIUS
R2 = RADIUS * RADIUS

NPAD = 10240
RB = 512
GB = 256
TBL_W = 128
SC_CHUNK = 128


def _kp_np():
    rng = np.random.RandomState(42)
    kp = rng.randn(KP, 3).astype(np.float32)
    kp /= (np.linalg.norm(kp, axis=1, keepdims=True) + 1e-9)
    kp *= rng.rand(KP, 1).astype(np.float32)
    kp[0] = 0.0
    return (kp * EXTENT).astype(np.float32)


_KPTS = _kp_np()


def _convin_body(x_ref, w_ref, b_ref, o_ref):
    o_ref[...] = (
        jnp.dot(x_ref[...], w_ref[...], preferred_element_type=jnp.float32)
        + b_ref[...]
    )


def _conv_in(x, wT, b2):
    return pl.pallas_call(
        _convin_body,
        grid=(NPAD // 1024,),
        in_specs=[
            pl.BlockSpec((1024, IN_CH), lambda i: (i, 0)),
            pl.BlockSpec((IN_CH, SC_CH), lambda i: (0, 0)),
            pl.BlockSpec((1, SC_CH), lambda i: (0, 0)),
        ],
        out_specs=pl.BlockSpec((1024, SC_CH), lambda i: (i, 0)),
        out_shape=jax.ShapeDtypeStruct((NPAD, SC_CH), jnp.float32),
    )(x, wT, b2)


NG = 128
GS = NPAD // NG


def _knn_body(cr_ref, ct_ref, o_ref):
    cr = cr_ref[...]
    ct = ct_ref[...]
    sqr = jnp.sum(cr * cr, axis=1, keepdims=True)
    sqc = jnp.sum(ct * ct, axis=0, keepdims=True)
    d2 = sqr + sqc - 2.0 * jnp.dot(cr, ct, preferred_element_type=jnp.float32)

    g = d2[:, :NG]
    for k in range(1, GS):
        g = jnp.minimum(g, d2[:, k * NG:(k + 1) * NG])
    lane = lax.broadcasted_iota(jnp.int32, (RB, NG), 1)
    gsel = []
    for _ in range(KNN):
        m = jnp.min(g, axis=1, keepdims=True)
        cand = jnp.where(g <= m, lane, jnp.int32(2**30))
        am = jnp.min(cand, axis=1, keepdims=True)
        gsel.append(am)
        g = jnp.where(cand == am, jnp.float32(1e30), g)
    g16 = jnp.concatenate(gsel, axis=1)

    idx128 = jnp.tile(g16, (1, NG // KNN))
    sec = lane // KNN
    cs, js = [], []
    for t in range(GS // 8):
        acc = None
        for s in range(8):
            k = 8 * t + s
            gth = jnp.take_along_axis(d2[:, k * NG:(k + 1) * NG], idx128, axis=1)
            acc = gth if acc is None else jnp.where(sec == s, gth, acc)
        cs.append(acc)
        js.append(NG * (8 * t + sec) + idx128)
    c = jnp.concatenate(cs, axis=1)
    ji = jnp.concatenate(js, axis=1)
    idxs = []
    for _ in range(KNN):
        m = jnp.min(c, axis=1, keepdims=True)
        cand = jnp.where(c <= m, ji, jnp.int32(2**30))
        am = jnp.min(cand, axis=1, keepdims=True)
        idxs.append(am)
        c = jnp.where(cand == am, jnp.float32(1e30), c)
    o_ref[...] = jnp.concatenate(idxs, axis=1)


def _knn(cp, ct):
    return pl.pallas_call(
        _knn_body,
        grid=(NPAD // RB,),
        in_specs=[
            pl.BlockSpec((RB, 8), lambda i: (i, 0)),
            pl.BlockSpec((8, NPAD), lambda i: (0, 0)),
        ],
        out_specs=pl.BlockSpec((RB, KNN), lambda i: (i, 0)),
        out_shape=jax.ShapeDtypeStruct((NPAD, KNN), jnp.int32),
    )(cp, ct)


SC_B = NPAD * KNN
SC_NW = 32


def _gather_sc(idx_flat, table):
    mesh = plsc.VectorSubcoreMesh(core_axis_name="c", subcore_axis_name="s")
    bpw = SC_B // SC_NW
    nch = bpw // SC_CHUNK

    @functools.partial(
        pl.kernel,
        out_type=jax.ShapeDtypeStruct((SC_B, TBL_W), jnp.float32),
        mesh=mesh,
        scratch_types=[
            pltpu.VMEM((SC_CHUNK,), jnp.int32),
            pltpu.VMEM((SC_CHUNK, TBL_W), jnp.float32),
            pltpu.SemaphoreType.DMA,
        ],
    )
    def gk(idx_hbm, tbl_hbm, out_hbm, idx_v, rows_v, sem):
        wid = lax.axis_index("s") * 2 + lax.axis_index("c")
        base = wid * bpw

        def body(j, carry):
            off = pl.multiple_of(base + j * SC_CHUNK, 8)
            pltpu.sync_copy(idx_hbm.at[pl.ds(off, SC_CHUNK)], idx_v)
            pltpu.async_copy(tbl_hbm.at[idx_v], rows_v, sem).wait()
            pltpu.sync_copy(rows_v, out_hbm.at[pl.ds(off, SC_CHUNK)])
            return carry

        lax.fori_loop(0, nch, body, 0)

    return gk(idx_flat, table)


GB2 = 4096
PW = 512

_KP16 = np.pad(_KPTS, ((0, 1), (0, 0)))
_SEL = (np.arange(PW)[None, :] // SC_CH == np.arange(KNN)[:, None]).astype(np.float32)
_FOLD = (np.arange(PW)[:, None] % SC_CH == np.arange(SC_CH)[None, :]).astype(np.float32)
_KC = np.zeros((8, PW), np.float32)
_KC[0, :KNN] = _KP16[:, 0]
_KC[1, :KNN] = _KP16[:, 1]
_KC[2, :KNN] = _KP16[:, 2]
_KC[3, :KP] = 1.0


def _agg_body(g_ref, cq_ref, w2_ref, fw_ref, bo_ref, kc_ref, sel_ref, y_ref):
    g = g_ref[...]
    nf = g[:, :SC_CH]
    f2 = jnp.dot(nf, w2_ref[...], preferred_element_type=jnp.float32)
    rx = g[:, SC_CH + 0:SC_CH + 1] - cq_ref[:, 0:1]
    ry = g[:, SC_CH + 1:SC_CH + 2] - cq_ref[:, 1:2]
    rz = g[:, SC_CH + 2:SC_CH + 3] - cq_ref[:, 2:3]
    nd2 = rx * rx + ry * ry + rz * rz
    maskf = (nd2 <= R2).astype(jnp.float32)
    dx = rx - kc_ref[0:1, :KNN]
    dy = ry - kc_ref[1:2, :KNN]
    dz = rz - kc_ref[2:3, :KNN]
    dist = jnp.sqrt(dx * dx + dy * dy + dz * dz + 1e-12)
    infl = jnp.maximum(0.0, 1.0 - dist / EXTENT) * maskf * kc_ref[3:4, :KNN]
    inflb = jnp.dot(infl, sel_ref[...], preferred_element_type=jnp.float32)
    z = inflb * f2
    zq = jnp.sum(z.reshape(GB2 // KNN, KNN, PW), axis=1)
    y_ref[...] = (
        jnp.dot(zq, fw_ref[...], preferred_element_type=jnp.float32)
        + bo_ref[...]
    )


def _agg(g2d, crep, w2all, fw, bo2):
    return pl.pallas_call(
        _agg_body,
        grid=(SC_B // GB2,),
        in_specs=[
            pl.BlockSpec((GB2, TBL_W), lambda i: (i, 0)),
            pl.BlockSpec((GB2, 8), lambda i: (i, 0)),
            pl.BlockSpec((SC_CH, PW), lambda i: (0, 0)),
            pl.BlockSpec((PW, OUT_CH), lambda i: (0, 0)),
            pl.BlockSpec((1, OUT_CH), lambda i: (0, 0)),
            pl.BlockSpec((8, PW), lambda i: (0, 0)),
            pl.BlockSpec((KNN, PW), lambda i: (0, 0)),
        ],
        out_specs=pl.BlockSpec((GB2 // KNN, OUT_CH), lambda i: (i, 0)),
        out_shape=jax.ShapeDtypeStruct((NPAD, OUT_CH), jnp.float32),
    )(g2d, crep, w2all, fw, bo2, jnp.asarray(_KC), jnp.asarray(_SEL))


def _bn_body(y_ref, g_ref, b_ref, o_ref):
    y = y_ref[...]
    rows = lax.broadcasted_iota(jnp.int32, (NPAD, 1), 0)
    validf = (rows < N).astype(jnp.float32)
    yv = y * validf
    mean = jnp.sum(yv, axis=0, keepdims=True) / N
    msq = jnp.sum(yv * yv, axis=0, keepdims=True) / N
    var = msq - mean * mean
    xh = (y - mean) / jnp.sqrt(var + 1e-5)
    out = xh * g_ref[...] + b_ref[...]
    o_ref[...] = jnp.where(out >= 0, out, 0.1 * out)


def _bn_lrelu(y, g2, b2):
    return pl.pallas_call(
        _bn_body,
        out_shape=jax.ShapeDtypeStruct((NPAD, OUT_CH), jnp.float32),
    )(y, g2, b2)


def _cloud(x, coords, w_inT, b_in2, w2all, fw, bo2, g2, be2):
    xt = jnp.pad(jnp.transpose(x[0]), ((0, NPAD - N), (0, 0)))
    cpad = jnp.concatenate(
        [coords[0], jnp.full((NPAD - N, 3), 1e3, jnp.float32)], axis=0
    )
    cp = jnp.pad(cpad, ((0, 0), (0, 5)))
    ct = jnp.transpose(cp)

    feats = _conv_in(xt, w_inT, b_in2)
    idx = _knn(cp, ct)
    table = jnp.concatenate(
        [feats, cp[:, :3], jnp.zeros((NPAD, TBL_W - SC_CH - 3), jnp.float32)],
        axis=1,
    )
    g = _gather_sc(idx.reshape(-1), table)
    crep = jnp.repeat(cp, KNN, axis=0)
    y = _agg(g, crep, w2all, fw, bo2)
    out = _bn_lrelu(y, g2, be2)
    return jnp.transpose(out[:N])[None]


def kernel(src, tgt, src_coords, tgt_coords, W_in, b_in, W_kp, W_out, b_out, gamma, beta):
    w_inT = jnp.transpose(W_in)
    b_in2 = b_in[None, :]
    w2all = jnp.pad(
        jnp.transpose(W_kp, (1, 0, 2)).reshape(SC_CH, KP * SC_CH),
        ((0, 0), (0, PW - KP * SC_CH)),
    )
    fw = jnp.dot(jnp.asarray(_FOLD), jnp.transpose(W_out))
    bo2 = b_out[None, :]
    g2 = gamma[None, :]
    be2 = beta[None, :]
    s3 = _cloud(src, src_coords, w_inT, b_in2, w2all, fw, bo2, g2, be2)
    t3 = _cloud(tgt, tgt_coords, w_inT, b_in2, w2all, fw, bo2, g2, be2)
    return (s3, t3, src_coords, tgt_coords)
